# baseline (speedup 1.0000x reference)
import numpy as np
import jax
import jax.numpy as jnp
from jax.experimental import pallas as pl
from jax.experimental.pallas import tpu as pltpu
from jax.experimental.pallas import tpu_sc as plsc

_B, _V = 128, 100000
_BC = 4096
_NB = (_V + _BC - 1) // _BC
_NH = (_NB + 1) // 2
_WC = 2 * _BC
_EPS = 1e-06


def _gumbel_table() -> np.ndarray:
    n = _B * _V
    ks0, ks1 = np.uint32(0), np.uint32(42)
    ks2 = np.uint32(ks0 ^ ks1 ^ np.uint32(0x1BD11BDA))
    ks = (ks0, ks1, ks2)
    rots = ((13, 15, 26, 6), (17, 29, 16, 24))
    x0 = np.full(n, ks0, dtype=np.uint32)
    x1 = (np.arange(n, dtype=np.uint32) + ks1).astype(np.uint32)
    for i in range(5):
        for r in rots[i % 2]:
            x0 = (x0 + x1).astype(np.uint32)
            x1 = ((x1 << np.uint32(r)) | (x1 >> np.uint32(32 - r))).astype(np.uint32)
            x1 ^= x0
        x0 = (x0 + ks[(i + 1) % 3]).astype(np.uint32)
        x1 = (x1 + ks[(i + 2) % 3] + np.uint32(i + 1)).astype(np.uint32)
    bits = x0 ^ x1
    u = ((bits >> np.uint32(9)) | np.uint32(0x3F800000)).view(np.float32) - np.float32(1.0)
    u = u * np.float32(0.999 - _EPS) + np.float32(_EPS)
    g = -np.log(-np.log(u))
    return g.reshape(_B, _V)


_G_TABLE = _gumbel_table()


def _scan_body(xl_ref, gl_ref, xr_ref, gr_ref, idx_ref, mx_ref, ix_ref):
    j = pl.program_id(0)
    jr = jnp.minimum(_NH + j, _NB - 1)

    def blockstat(v, base_col):
        col = jax.lax.broadcasted_iota(jnp.int32, v.shape, 1) + base_col
        v = jnp.where(col < _V, v, -jnp.inf)
        bmax = jnp.max(v, axis=1, keepdims=True)
        bidx = jnp.min(jnp.where(v == bmax, col, jnp.int32(2**31 - 1)),
                       axis=1, keepdims=True)
        return bmax, bidx

    lmax, lidx = blockstat(xl_ref[...] + gl_ref[...], j * _BC)
    rmax, ridx = blockstat(xr_ref[...] + gr_ref[...], jr * _BC)
    take_r = rmax > lmax
    bmax = jnp.where(take_r, rmax, lmax)
    bidx = jnp.where(take_r, ridx, lidx)

    @pl.when(j == 0)
    def _():
        mx_ref[...] = bmax
        ix_ref[...] = bidx

    @pl.when(j > 0)
    def _():
        better = bmax > mx_ref[...]
        mx_ref[...] = jnp.where(better, bmax, mx_ref[...])
        ix_ref[...] = jnp.where(better, bidx, ix_ref[...])

    @pl.when(j == _NH - 1)
    def _():
        idx_ref[...] = ix_ref[...]


_NWORK = 32
_WSPAN = (_B * _V) // _NWORK
_ZCH = 20000
_NCH = _WSPAN // _ZCH


def _sc_zero_body(out_ref, zbuf, sem):
    wid = jax.lax.axis_index("s") * 2 + jax.lax.axis_index("c")

    def zb(i, carry):
        zbuf[pl.ds(i * 16, 16)] = jnp.zeros((16,), jnp.float32)
        return carry

    jax.lax.fori_loop(0, _ZCH // 16, zb, 0)
    base = wid * _WSPAN
    copies = []
    for k in range(_NCH):
        cp = pltpu.make_async_copy(
            zbuf, out_ref.at[pl.ds(base + k * _ZCH, _ZCH)], sem)
        cp.start()
        copies.append(cp)
    for cp in copies:
        cp.wait()


def _patch_body(idx_s, idx_v, zsrc, out_ref, tile_ref, sem):
    del zsrc
    lane = jax.lax.broadcasted_iota(jnp.int32, (8, 128), 1)
    copies = []
    for r in range(_B):
        a = r // 8
        base = pl.multiple_of((idx_s[r, 0] // 128) * 128, 128)
        gvals = idx_v[pl.ds(8 * a, 8), :]
        tile_ref[pl.ds(8 * r, 8), :] = (gvals == base + lane).astype(jnp.float32)
        cp = pltpu.make_async_copy(
            tile_ref.at[pl.ds(8 * r, 8), :],
            out_ref.at[pl.ds(8 * a, 8), pl.ds(base, 128)],
            sem)
        cp.start()
        copies.append(cp)
    for cp in copies:
        cp.wait()


def kernel(logits, param, W):
    g = jnp.asarray(_G_TABLE)
    half_idx = lambda j: (0, jnp.minimum(_NH + j, _NB - 1))
    zeros = pl.kernel(
        _sc_zero_body,
        out_type=jax.ShapeDtypeStruct((_B * _V,), jnp.float32),
        mesh=plsc.VectorSubcoreMesh(core_axis_name="c", subcore_axis_name="s"),
        scratch_types=[pltpu.VMEM((_ZCH,), jnp.float32),
                       pltpu.SemaphoreType.DMA],
    )().reshape(_B, _V)
    idx = pl.pallas_call(
        _scan_body,
        grid=(_NH,),
        in_specs=[pl.BlockSpec((_B, _BC), lambda j: (0, j)),
                  pl.BlockSpec((_B, _BC), lambda j: (0, j)),
                  pl.BlockSpec((_B, _BC), half_idx),
                  pl.BlockSpec((_B, _BC), half_idx)],
        out_specs=pl.BlockSpec((_B, 1), lambda j: (0, 0)),
        out_shape=jax.ShapeDtypeStruct((_B, 1), jnp.int32),
        scratch_shapes=[pltpu.VMEM((_B, 1), jnp.float32),
                        pltpu.VMEM((_B, 1), jnp.int32)],
    )(logits, g, logits, g)
    out = pl.pallas_call(
        _patch_body,
        in_specs=[pl.BlockSpec(memory_space=pltpu.SMEM),
                  pl.BlockSpec(memory_space=pltpu.VMEM),
                  pl.BlockSpec(memory_space=pl.ANY)],
        out_specs=pl.BlockSpec(memory_space=pl.ANY),
        out_shape=jax.ShapeDtypeStruct((_B, _V), jnp.float32),
        scratch_shapes=[pltpu.VMEM((8 * _B, 128), jnp.float32),
                        pltpu.SemaphoreType.DMA],
        input_output_aliases={2: 0},
    )(idx, idx, zeros)
    return out


# speedup vs baseline: 1.5726x; 1.5726x over previous
import numpy as np
import jax
import jax.numpy as jnp
from jax.experimental import pallas as pl
from jax.experimental.pallas import tpu as pltpu

_B, _V = 128, 100000
_BC = 4096
_NB = (_V + _BC - 1) // _BC
_NH = (_NB + 1) // 2
_WC = 2 * _BC
_EPS = 1e-06


def _gumbel_table() -> np.ndarray:
    n = _B * _V
    ks0, ks1 = np.uint32(0), np.uint32(42)
    ks2 = np.uint32(ks0 ^ ks1 ^ np.uint32(0x1BD11BDA))
    ks = (ks0, ks1, ks2)
    rots = ((13, 15, 26, 6), (17, 29, 16, 24))
    x0 = np.full(n, ks0, dtype=np.uint32)
    x1 = (np.arange(n, dtype=np.uint32) + ks1).astype(np.uint32)
    for i in range(5):
        for r in rots[i % 2]:
            x0 = (x0 + x1).astype(np.uint32)
            x1 = ((x1 << np.uint32(r)) | (x1 >> np.uint32(32 - r))).astype(np.uint32)
            x1 ^= x0
        x0 = (x0 + ks[(i + 1) % 3]).astype(np.uint32)
        x1 = (x1 + ks[(i + 2) % 3] + np.uint32(i + 1)).astype(np.uint32)
    bits = x0 ^ x1
    u = ((bits >> np.uint32(9)) | np.uint32(0x3F800000)).view(np.float32) - np.float32(1.0)
    u = u * np.float32(0.999 - _EPS) + np.float32(_EPS)
    g = -np.log(-np.log(u))
    return g.reshape(_B, _V)


_G_TABLE = _gumbel_table()


def _scan_body(xl_ref, gl_ref, xr_ref, gr_ref, z_ref, idx_ref, mx_ref, ix_ref):
    j = pl.program_id(0)
    jr = jnp.minimum(_NH + j, _NB - 1)

    def blockstat(v, base_col):
        col = jax.lax.broadcasted_iota(jnp.int32, v.shape, 1) + base_col
        v = jnp.where(col < _V, v, -jnp.inf)
        bmax = jnp.max(v, axis=1, keepdims=True)
        bidx = jnp.min(jnp.where(v == bmax, col, jnp.int32(2**31 - 1)),
                       axis=1, keepdims=True)
        return bmax, bidx

    lmax, lidx = blockstat(xl_ref[...] + gl_ref[...], j * _BC)
    rmax, ridx = blockstat(xr_ref[...] + gr_ref[...], jr * _BC)
    take_r = rmax > lmax
    bmax = jnp.where(take_r, rmax, lmax)
    bidx = jnp.where(take_r, ridx, lidx)

    z_ref[...] = jnp.zeros_like(z_ref)

    @pl.when(j == 0)
    def _():
        mx_ref[...] = bmax
        ix_ref[...] = bidx

    @pl.when(j > 0)
    def _():
        better = bmax > mx_ref[...]
        mx_ref[...] = jnp.where(better, bmax, mx_ref[...])
        ix_ref[...] = jnp.where(better, bidx, ix_ref[...])

    @pl.when(j == _NH - 1)
    def _():
        idx_ref[...] = ix_ref[...]


def _patch_body(idx_s, idx_v, zsrc, out_ref, tile_ref, sem):
    del zsrc
    lane = jax.lax.broadcasted_iota(jnp.int32, (8, 128), 1)
    copies = []
    for r in range(_B):
        a = r // 8
        base = pl.multiple_of((idx_s[r, 0] // 128) * 128, 128)
        gvals = idx_v[pl.ds(8 * a, 8), :]
        tile_ref[pl.ds(8 * r, 8), :] = (gvals == base + lane).astype(jnp.float32)
        cp = pltpu.make_async_copy(
            tile_ref.at[pl.ds(8 * r, 8), :],
            out_ref.at[pl.ds(8 * a, 8), pl.ds(base, 128)],
            sem)
        cp.start()
        copies.append(cp)
    for cp in copies:
        cp.wait()


def kernel(logits, param, W):
    g = jnp.asarray(_G_TABLE)
    half_idx = lambda j: (0, jnp.minimum(_NH + j, _NB - 1))
    zidx = pl.pallas_call(
        _scan_body,
        grid=(_NH,),
        in_specs=[pl.BlockSpec((_B, _BC), lambda j: (0, j)),
                  pl.BlockSpec((_B, _BC), lambda j: (0, j)),
                  pl.BlockSpec((_B, _BC), half_idx),
                  pl.BlockSpec((_B, _BC), half_idx)],
        out_specs=[pl.BlockSpec((_B, _WC), lambda j: (0, j)),
                   pl.BlockSpec((_B, 1), lambda j: (0, 0))],
        out_shape=[jax.ShapeDtypeStruct((_B, _V), jnp.float32),
                   jax.ShapeDtypeStruct((_B, 1), jnp.int32)],
        scratch_shapes=[pltpu.VMEM((_B, 1), jnp.float32),
                        pltpu.VMEM((_B, 1), jnp.int32)],
    )(logits, g, logits, g)
    zeros = zidx[0]
    idx = zidx[1]
    out = pl.pallas_call(
        _patch_body,
        in_specs=[pl.BlockSpec(memory_space=pltpu.SMEM),
                  pl.BlockSpec(memory_space=pltpu.VMEM),
                  pl.BlockSpec(memory_space=pl.ANY)],
        out_specs=pl.BlockSpec(memory_space=pl.ANY),
        out_shape=jax.ShapeDtypeStruct((_B, _V), jnp.float32),
        scratch_shapes=[pltpu.VMEM((8 * _B, 128), jnp.float32),
                        pltpu.SemaphoreType.DMA],
        input_output_aliases={2: 0},
    )(idx, idx, zeros)
    return out


# speedup vs baseline: 1.5851x; 1.0079x over previous
import numpy as np
import jax
import jax.numpy as jnp
from jax.experimental import pallas as pl
from jax.experimental.pallas import tpu as pltpu

_B, _V = 128, 100000
_BC = 4096
_NB = (_V + _BC - 1) // _BC
_NQ = (_NB + 3) // 4
_WC = 4 * _BC
_EPS = 1e-06


def _gumbel_table() -> np.ndarray:
    n = _B * _V
    ks0, ks1 = np.uint32(0), np.uint32(42)
    ks2 = np.uint32(ks0 ^ ks1 ^ np.uint32(0x1BD11BDA))
    ks = (ks0, ks1, ks2)
    rots = ((13, 15, 26, 6), (17, 29, 16, 24))
    x0 = np.full(n, ks0, dtype=np.uint32)
    x1 = (np.arange(n, dtype=np.uint32) + ks1).astype(np.uint32)
    for i in range(5):
        for r in rots[i % 2]:
            x0 = (x0 + x1).astype(np.uint32)
            x1 = ((x1 << np.uint32(r)) | (x1 >> np.uint32(32 - r))).astype(np.uint32)
            x1 ^= x0
        x0 = (x0 + ks[(i + 1) % 3]).astype(np.uint32)
        x1 = (x1 + ks[(i + 2) % 3] + np.uint32(i + 1)).astype(np.uint32)
    bits = x0 ^ x1
    u = ((bits >> np.uint32(9)) | np.uint32(0x3F800000)).view(np.float32) - np.float32(1.0)
    u = u * np.float32(0.999 - _EPS) + np.float32(_EPS)
    g = -np.log(-np.log(u))
    return g.reshape(_B, _V)


_G_TABLE = _gumbel_table()


def _scan_body(x0, g0, x1, g1, x2, g2, x3, g3, z_ref, idx_ref, mx_ref, ix_ref):
    j = pl.program_id(0)

    def blockstat(v, base_col):
        col = jax.lax.broadcasted_iota(jnp.int32, v.shape, 1) + base_col
        v = jnp.where(col < _V, v, -jnp.inf)
        bmax = jnp.max(v, axis=1, keepdims=True)
        bidx = jnp.min(jnp.where(v == bmax, col, jnp.int32(2**31 - 1)),
                       axis=1, keepdims=True)
        return bmax, bidx

    bmax = None
    for q, (xr, gr) in enumerate(((x0, g0), (x1, g1), (x2, g2), (x3, g3))):
        bq = jnp.minimum(q * _NQ + j, _NB - 1)
        qmax, qidx = blockstat(xr[...] + gr[...], bq * _BC)
        if bmax is None:
            bmax, bidx = qmax, qidx
        else:
            take_q = (qmax > bmax) | ((qmax == bmax) & (qidx < bidx))
            bmax = jnp.where(take_q, qmax, bmax)
            bidx = jnp.where(take_q, qidx, bidx)

    z_ref[...] = jnp.zeros_like(z_ref)

    @pl.when(j == 0)
    def _():
        mx_ref[...] = bmax
        ix_ref[...] = bidx

    @pl.when(j > 0)
    def _():
        better = bmax > mx_ref[...]
        mx_ref[...] = jnp.where(better, bmax, mx_ref[...])
        ix_ref[...] = jnp.where(better, bidx, ix_ref[...])

    @pl.when(j == _NQ - 1)
    def _():
        idx_ref[...] = ix_ref[...]


def _patch_body(idx_s, idx_v, zsrc, out_ref, tile_ref, sem):
    del zsrc
    lane = jax.lax.broadcasted_iota(jnp.int32, (8, 128), 1)
    copies = []
    for r in range(_B):
        a = r // 8
        base = pl.multiple_of((idx_s[r, 0] // 128) * 128, 128)
        gvals = idx_v[pl.ds(8 * a, 8), :]
        tile_ref[pl.ds(8 * r, 8), :] = (gvals == base + lane).astype(jnp.float32)
        cp = pltpu.make_async_copy(
            tile_ref.at[pl.ds(8 * r, 8), :],
            out_ref.at[pl.ds(8 * a, 8), pl.ds(base, 128)],
            sem)
        cp.start()
        copies.append(cp)
    for cp in copies:
        cp.wait()


def kernel(logits, param, W):
    g = jnp.asarray(_G_TABLE)
    qspec = [pl.BlockSpec((_B, _BC), (lambda q: (lambda j: (0, jnp.minimum(q * _NQ + j, _NB - 1))))(q))
             for q in range(4) for _ in range(2)]
    zidx = pl.pallas_call(
        _scan_body,
        grid=(_NQ,),
        in_specs=qspec,
        out_specs=[pl.BlockSpec((_B, _WC), lambda j: (0, j)),
                   pl.BlockSpec((_B, 1), lambda j: (0, 0))],
        out_shape=[jax.ShapeDtypeStruct((_B, _V), jnp.float32),
                   jax.ShapeDtypeStruct((_B, 1), jnp.int32)],
        scratch_shapes=[pltpu.VMEM((_B, 1), jnp.float32),
                        pltpu.VMEM((_B, 1), jnp.int32)],
    )(logits, g, logits, g, logits, g, logits, g)
    zeros = zidx[0]
    idx = zidx[1]
    out = pl.pallas_call(
        _patch_body,
        in_specs=[pl.BlockSpec(memory_space=pltpu.SMEM),
                  pl.BlockSpec(memory_space=pltpu.VMEM),
                  pl.BlockSpec(memory_space=pl.ANY)],
        out_specs=pl.BlockSpec(memory_space=pl.ANY),
        out_shape=jax.ShapeDtypeStruct((_B, _V), jnp.float32),
        scratch_shapes=[pltpu.VMEM((8 * _B, 128), jnp.float32),
                        pltpu.SemaphoreType.DMA],
        input_output_aliases={2: 0},
    )(idx, idx, zeros)
    return out
